# baseline (speedup 1.0000x reference)
import functools

import jax
import jax.numpy as jnp
from jax import lax
from jax.experimental import pallas as pl
from jax.experimental.pallas import tpu as pltpu
from jax.experimental.pallas import tpu_sc as plsc

N_FIELDS = 26
BATCH = 16384
VOCAB = 100000
EMBED_DIM = 128

_NC = 2
_NS = 16
_NW = _NC * _NS
_BPW = BATCH // _NW
_BC = 16
_NCHUNK = _BPW // _BC
_ROWS = _BC * N_FIELDS
_NSTREAM = 4
_RPS = _ROWS // _NSTREAM


def _body(tab_hbm, ids_hbm, out_hbm, ids_v, fused_v, rows_v, gsem):
    c = lax.axis_index("c")
    s = lax.axis_index("s")
    wid = s * _NC + c
    base = wid * _BPW

    pltpu.sync_copy(ids_hbm.at[:, pl.ds(base * 1, _BPW)], ids_v)

    iota26 = lax.iota(jnp.int32, 16) * N_FIELDS

    def chunk(ci, carry):
        cb = ci * _BC
        for f in range(N_FIELDS):
            vec = ids_v[f, pl.ds(cb, 16)]
            plsc.store_scatter(fused_v, [iota26 + f], vec + f * VOCAB)
        cps = []
        for j in range(_NSTREAM):
            cps.append(
                pltpu.async_copy(
                    tab_hbm.at[fused_v.at[pl.ds(j * _RPS, _RPS)]],
                    rows_v.at[pl.ds(j * _RPS, _RPS)],
                    gsem,
                )
            )
        for cp in cps:
            cp.wait()
        pltpu.sync_copy(rows_v, out_hbm.at[pl.ds((base + cb) * N_FIELDS, _ROWS)])
        return carry

    lax.fori_loop(0, _NCHUNK, chunk, 0)


@jax.jit
def _lookup(tab_flat, ids32):
    run = pl.kernel(
        _body,
        out_type=jax.ShapeDtypeStruct((BATCH * N_FIELDS, EMBED_DIM), jnp.float32),
        mesh=plsc.VectorSubcoreMesh(core_axis_name="c", subcore_axis_name="s"),
        compiler_params=pltpu.CompilerParams(needs_layout_passes=False),
        scratch_types=[
            pltpu.VMEM((N_FIELDS, _BPW), jnp.int32),
            pltpu.VMEM((_ROWS,), jnp.int32),
            pltpu.VMEM((_ROWS, EMBED_DIM), jnp.float32),
            pltpu.SemaphoreType.DMA,
        ],
    )
    return run(tab_flat, ids32)


def kernel(ids, tables):
    ids32 = ids.astype(jnp.int32)
    tab_flat = tables.reshape(N_FIELDS * VOCAB, EMBED_DIM)
    out = _lookup(tab_flat, ids32)
    return out.reshape(BATCH, N_FIELDS, EMBED_DIM)


# speedup vs baseline: 1.0260x; 1.0260x over previous
import jax
import jax.numpy as jnp
from jax import lax
from jax.experimental import pallas as pl
from jax.experimental.pallas import tpu as pltpu
from jax.experimental.pallas import tpu_sc as plsc

N_FIELDS = 26
BATCH = 16384
VOCAB = 100000
EMBED_DIM = 128

_NC = 2
_NS = 16
_NW = _NC * _NS
_BPW = BATCH // _NW
_BC = 16
_NCHUNK = _BPW // _BC
_ROWS = _BC * N_FIELDS
_NSTREAM = 4
_RPS = _ROWS // _NSTREAM


def _body(tab_hbm, ids_hbm, out_hbm, ids_v, fused0, fused1, rows0, rows1, gsem, osem):
    fused_bufs = (fused0, fused1)
    rows_bufs = (rows0, rows1)
    c = lax.axis_index("c")
    s = lax.axis_index("s")
    wid = s * _NC + c
    base = wid * _BPW

    pltpu.sync_copy(ids_hbm.at[:, pl.ds(base, _BPW)], ids_v)

    iota26 = lax.iota(jnp.int32, 16) * N_FIELDS

    def run_chunk(ci, buf, drain_out):
        cb = ci * _BC
        fused_v = fused_bufs[buf]
        rows_v = rows_bufs[buf]
        for f in range(N_FIELDS):
            vec = ids_v[f, pl.ds(cb, 16)]
            plsc.store_scatter(fused_v, [iota26 + f], vec + f * VOCAB)
        if drain_out:
            pltpu.make_async_copy(
                out_hbm.at[pl.ds(0, _ROWS)], rows_v, osem
            ).wait()
        cps = [
            pltpu.async_copy(
                tab_hbm.at[fused_v.at[pl.ds(j * _RPS, _RPS)]],
                rows_v.at[pl.ds(j * _RPS, _RPS)],
                gsem,
            )
            for j in range(_NSTREAM)
        ]
        for cp in cps:
            cp.wait()
        pltpu.async_copy(
            rows_v, out_hbm.at[pl.ds((base + cb) * N_FIELDS, _ROWS)], osem
        )

    run_chunk(0, 0, drain_out=False)
    run_chunk(1, 1, drain_out=False)

    def step(i, carry):
        ci = 2 * i + 2
        run_chunk(ci, 0, drain_out=True)
        run_chunk(ci + 1, 1, drain_out=True)
        return carry

    lax.fori_loop(0, (_NCHUNK - 2) // 2, step, 0)

    for rv in rows_bufs:
        pltpu.make_async_copy(out_hbm.at[pl.ds(0, _ROWS)], rv, osem).wait()


@jax.jit
def _lookup(tab_flat, ids32):
    run = pl.kernel(
        _body,
        out_type=jax.ShapeDtypeStruct((BATCH * N_FIELDS, EMBED_DIM), jnp.float32),
        mesh=plsc.VectorSubcoreMesh(core_axis_name="c", subcore_axis_name="s"),
        compiler_params=pltpu.CompilerParams(needs_layout_passes=False),
        scratch_types=[
            pltpu.VMEM((N_FIELDS, _BPW), jnp.int32),
            pltpu.VMEM((_ROWS,), jnp.int32),
            pltpu.VMEM((_ROWS,), jnp.int32),
            pltpu.VMEM((_ROWS, EMBED_DIM), jnp.float32),
            pltpu.VMEM((_ROWS, EMBED_DIM), jnp.float32),
            pltpu.SemaphoreType.DMA,
            pltpu.SemaphoreType.DMA,
        ],
    )
    return run(tab_flat, ids32)


def kernel(ids, tables):
    ids32 = ids.astype(jnp.int32)
    tab_flat = tables.reshape(N_FIELDS * VOCAB, EMBED_DIM)
    out = _lookup(tab_flat, ids32)
    return out.reshape(BATCH, N_FIELDS, EMBED_DIM)


# speedup vs baseline: 3.4132x; 3.3266x over previous
import jax
import jax.numpy as jnp
from jax import lax
from jax.experimental import pallas as pl
from jax.experimental.pallas import tpu as pltpu
from jax.experimental.pallas import tpu_sc as plsc

N_FIELDS = 26
BATCH = 16384
VOCAB = 100000
EMBED_DIM = 128

_NC = 2
_NS = 16
_NW = _NC * _NS
_ROWS_TOTAL = N_FIELDS * BATCH
_RPW = _ROWS_TOTAL // _NW
_RC = 256
_NCHUNK = _RPW // _RC
_NSTREAM = 2
_RPS = _RC // _NSTREAM


def _body(tab_hbm, ids_hbm, out_hbm, ids_v, fused0, fused1, rows0, rows1, gsem, osem):
    c = lax.axis_index("c")
    s = lax.axis_index("s")
    wid = s * _NC + c
    wbase = wid * _RPW

    pltpu.sync_copy(ids_hbm.at[pl.ds(wbase, _RPW)], ids_v)

    fused_bufs = (fused0, fused1)
    rows_bufs = (rows0, rows1)

    def run_chunk(ci, buf, drain_out):
        cb = ci * _RC
        fused_v = fused_bufs[buf]
        rows_v = rows_bufs[buf]
        off = ((wbase + cb) >> 14) * VOCAB
        for v in range(_RC // 16):
            fused_v[pl.ds(v * 16, 16)] = ids_v[pl.ds(cb + v * 16, 16)] + off
        if drain_out:
            pltpu.make_async_copy(
                out_hbm.at[pl.ds(0, _RC)], rows_v, osem
            ).wait()
        cps = [
            pltpu.async_copy(
                tab_hbm.at[fused_v.at[pl.ds(j * _RPS, _RPS)]],
                rows_v.at[pl.ds(j * _RPS, _RPS)],
                gsem,
            )
            for j in range(_NSTREAM)
        ]
        for cp in cps:
            cp.wait()
        pltpu.async_copy(
            rows_v, out_hbm.at[pl.ds(wbase + cb, _RC)], osem
        )

    run_chunk(0, 0, drain_out=False)
    run_chunk(1, 1, drain_out=False)

    def step(i, carry):
        ci = 2 * i + 2
        run_chunk(ci, 0, drain_out=True)
        run_chunk(ci + 1, 1, drain_out=True)
        return carry

    lax.fori_loop(0, (_NCHUNK - 2) // 2, step, 0)

    for rv in rows_bufs:
        pltpu.make_async_copy(out_hbm.at[pl.ds(0, _RC)], rv, osem).wait()


@jax.jit
def _lookup(tab_flat, ids_flat):
    run = pl.kernel(
        _body,
        out_type=jax.ShapeDtypeStruct((_ROWS_TOTAL, EMBED_DIM), jnp.float32),
        mesh=plsc.VectorSubcoreMesh(core_axis_name="c", subcore_axis_name="s"),
        compiler_params=pltpu.CompilerParams(needs_layout_passes=False),
        scratch_types=[
            pltpu.VMEM((_RPW,), jnp.int32),
            pltpu.VMEM((_RC,), jnp.int32),
            pltpu.VMEM((_RC,), jnp.int32),
            pltpu.VMEM((_RC, EMBED_DIM), jnp.float32),
            pltpu.VMEM((_RC, EMBED_DIM), jnp.float32),
            pltpu.SemaphoreType.DMA,
            pltpu.SemaphoreType.DMA,
        ],
    )
    return run(tab_flat, ids_flat)


def kernel(ids, tables):
    ids_flat = ids.astype(jnp.int32).reshape(_ROWS_TOTAL)
    tab_flat = tables.reshape(N_FIELDS * VOCAB, EMBED_DIM)
    out = _lookup(tab_flat, ids_flat)
    return jnp.transpose(out.reshape(N_FIELDS, BATCH, EMBED_DIM), (1, 0, 2))


# speedup vs baseline: 3.5832x; 1.0498x over previous
import jax
import jax.numpy as jnp
from jax import lax
from jax.experimental import pallas as pl
from jax.experimental.pallas import tpu as pltpu
from jax.experimental.pallas import tpu_sc as plsc

N_FIELDS = 26
BATCH = 16384
VOCAB = 100000
EMBED_DIM = 128

_NC = 2
_NS = 16
_NW = _NC * _NS
_ROWS_TOTAL = N_FIELDS * BATCH
_RPW = _ROWS_TOTAL // _NW
_RC = 128
_NCHUNK = _RPW // _RC
_NBUF = 4


def _body(tab_hbm, ids_hbm, out_hbm, ids_v, f0, f1, f2, f3, r0, r1, r2, r3,
          gsem, osem):
    c = lax.axis_index("c")
    s = lax.axis_index("s")
    wid = s * _NC + c
    wbase = wid * _RPW

    pltpu.sync_copy(ids_hbm.at[pl.ds(wbase, _RPW)], ids_v)

    fused = (f0, f1, f2, f3)
    rows = (r0, r1, r2, r3)

    def fire_gather(ci, b):
        off = ((wbase + ci * _RC) >> 14) * VOCAB
        for v in range(_RC // 16):
            fused[b][pl.ds(v * 16, 16)] = ids_v[pl.ds(ci * _RC + v * 16, 16)] + off
        pltpu.async_copy(tab_hbm.at[fused[b]], rows[b], gsem)

    def wait_gather(b):
        pltpu.make_async_copy(tab_hbm.at[fused[b]], rows[b], gsem).wait()

    def fire_out(ci, b):
        pltpu.async_copy(rows[b], out_hbm.at[pl.ds(wbase + ci * _RC, _RC)], osem)

    def drain_out(b):
        pltpu.make_async_copy(out_hbm.at[pl.ds(0, _RC)], rows[b], osem).wait()

    fire_gather(0, 0)
    fire_gather(1, 1)
    for b in range(_NBUF):
        wait_gather(b)
        fire_out(b, b)
        if b >= 2:
            drain_out((b + 2) % _NBUF)
        fire_gather(b + 2, (b + 2) % _NBUF)

    def step(i, carry):
        ci = _NBUF * i + _NBUF
        for b in range(_NBUF):
            wait_gather(b)
            fire_out(ci + b, b)
            drain_out((b + 2) % _NBUF)
            fire_gather(ci + b + 2, (b + 2) % _NBUF)
        return carry

    lax.fori_loop(0, (_NCHUNK - 2 * _NBUF) // _NBUF, step, 0)

    last = _NCHUNK - _NBUF
    for b in range(_NBUF):
        wait_gather(b)
        fire_out(last + b, b)
        if b < 2:
            drain_out((b + 2) % _NBUF)
            fire_gather(last + b + 2, (b + 2) % _NBUF)
    for b in range(_NBUF):
        drain_out(b)


@jax.jit
def _lookup(tab_flat, ids_flat):
    run = pl.kernel(
        _body,
        out_type=jax.ShapeDtypeStruct((_ROWS_TOTAL, EMBED_DIM), jnp.float32),
        mesh=plsc.VectorSubcoreMesh(core_axis_name="c", subcore_axis_name="s"),
        compiler_params=pltpu.CompilerParams(needs_layout_passes=False),
        scratch_types=[
            pltpu.VMEM((_RPW,), jnp.int32),
            pltpu.VMEM((_RC,), jnp.int32),
            pltpu.VMEM((_RC,), jnp.int32),
            pltpu.VMEM((_RC,), jnp.int32),
            pltpu.VMEM((_RC,), jnp.int32),
            pltpu.VMEM((_RC, EMBED_DIM), jnp.float32),
            pltpu.VMEM((_RC, EMBED_DIM), jnp.float32),
            pltpu.VMEM((_RC, EMBED_DIM), jnp.float32),
            pltpu.VMEM((_RC, EMBED_DIM), jnp.float32),
            pltpu.SemaphoreType.DMA,
            pltpu.SemaphoreType.DMA,
        ],
    )
    return run(tab_flat, ids_flat)


def kernel(ids, tables):
    ids_flat = ids.astype(jnp.int32).reshape(_ROWS_TOTAL)
    tab_flat = tables.reshape(N_FIELDS * VOCAB, EMBED_DIM)
    out = _lookup(tab_flat, ids_flat)
    return jnp.transpose(out.reshape(N_FIELDS, BATCH, EMBED_DIM), (1, 0, 2))
